# Initial kernel scaffold; baseline (speedup 1.0000x reference)
import jax
import jax.numpy as jnp
from jax.experimental import pallas as pl


def kernel(input_ids, tok_emb):
    raise NotImplementedError("write your pallas kernel here")


# speedup vs baseline: 1.2827x; 1.2827x over previous
import functools

import jax
import jax.numpy as jnp
from jax import lax
from jax.experimental import pallas as pl
from jax.experimental.pallas import tpu as pltpu
from jax.experimental.pallas import tpu_sc as plsc

VOCAB = 1000000
HIDDEN = 128

B_TOTAL = 4096 * 200
CHUNK = 128
N_ROWS = B_TOTAL // CHUNK
NC, NS = 2, 16
NW = NC * NS
ROWS_PER_W = N_ROWS // NW


def _make_gather():
  mesh = plsc.VectorSubcoreMesh(core_axis_name="c", subcore_axis_name="s")

  @functools.partial(
      pl.kernel,
      out_type=jax.ShapeDtypeStruct((B_TOTAL, HIDDEN), jnp.float32),
      mesh=mesh,
      scratch_types=[
          pltpu.VMEM((ROWS_PER_W, CHUNK), jnp.int32),
          pltpu.VMEM((CHUNK, HIDDEN), jnp.float32),
          pltpu.SemaphoreType.DMA,
      ],
  )
  def gather_kernel(idx_hbm, tab_hbm, out_hbm, idx_v, rows_v, sem):
    wid = lax.axis_index("s") * NC + lax.axis_index("c")
    row_base = wid * ROWS_PER_W
    pltpu.sync_copy(idx_hbm.at[pl.ds(row_base, ROWS_PER_W)], idx_v)

    @pl.loop(0, ROWS_PER_W)
    def _chunk(j):
      pltpu.async_copy(tab_hbm.at[idx_v.at[j]], rows_v, sem).wait()
      pltpu.sync_copy(
          rows_v, out_hbm.at[pl.ds((row_base + j) * CHUNK, CHUNK)])

  return gather_kernel


_gather = _make_gather()


@jax.jit
def kernel(input_ids, tok_emb):
  idx = input_ids.reshape(N_ROWS, CHUNK).astype(jnp.int32)
  out = _gather(idx, tok_emb)
  return out.reshape(input_ids.shape[0], input_ids.shape[1], HIDDEN)


# speedup vs baseline: 1.8486x; 1.4412x over previous
import functools

import jax
import jax.numpy as jnp
from jax import lax
from jax.experimental import pallas as pl
from jax.experimental.pallas import tpu as pltpu
from jax.experimental.pallas import tpu_sc as plsc

VOCAB = 1000000
HIDDEN = 128

B_TOTAL = 4096 * 200
CHUNK = 128
N_ROWS = B_TOTAL // CHUNK
NC, NS = 2, 16
NW = NC * NS
ROWS_PER_W = N_ROWS // NW
NBUF = 4
LOOK = 2


def _make_gather():
  mesh = plsc.VectorSubcoreMesh(core_axis_name="c", subcore_axis_name="s")

  @functools.partial(
      pl.kernel,
      out_type=jax.ShapeDtypeStruct((B_TOTAL, HIDDEN), jnp.float32),
      mesh=mesh,
      scratch_types=[
          pltpu.VMEM((ROWS_PER_W, CHUNK), jnp.int32),
          pltpu.VMEM((NBUF, CHUNK, HIDDEN), jnp.float32),
          pltpu.SemaphoreType.DMA,
          pltpu.SemaphoreType.DMA,
          pltpu.SemaphoreType.DMA,
          pltpu.SemaphoreType.DMA,
          pltpu.SemaphoreType.DMA,
          pltpu.SemaphoreType.DMA,
          pltpu.SemaphoreType.DMA,
          pltpu.SemaphoreType.DMA,
      ],
  )
  def gather_kernel(idx_hbm, tab_hbm, out_hbm, idx_v, rows_v,
                    g0, g1, g2, g3, s0, s1, s2, s3):
    gsems = [g0, g1, g2, g3]
    ssem = [s0, s1, s2, s3]
    wid = lax.axis_index("s") * NC + lax.axis_index("c")
    row_base = wid * ROWS_PER_W
    pltpu.sync_copy(idx_hbm.at[pl.ds(row_base, ROWS_PER_W)], idx_v)

    for k in range(LOOK):
      pltpu.async_copy(tab_hbm.at[idx_v.at[k]], rows_v.at[k], gsems[k])

    @pl.loop(0, ROWS_PER_W, step=NBUF)
    def _chunk(j0):
      for b in range(NBUF):
        j = j0 + b
        buf = rows_v.at[b]
        pltpu.make_async_copy(tab_hbm.at[idx_v.at[j]], buf, gsems[b]).wait()
        out_slice = out_hbm.at[pl.ds((row_base + j) * CHUNK, CHUNK)]
        pltpu.async_copy(buf, out_slice, ssem[b])

        k = j + LOOK
        bk = (b + LOOK) % NBUF
        nxt = rows_v.at[bk]

        @pl.when(k < ROWS_PER_W)
        def _():
          @pl.when(k >= NBUF)
          def _():
            pltpu.make_async_copy(
                nxt, out_hbm.at[pl.ds(0, CHUNK)], ssem[bk]).wait()

          pltpu.async_copy(tab_hbm.at[idx_v.at[k]], nxt, gsems[bk])

    for b in range(NBUF):
      pltpu.make_async_copy(
          rows_v.at[b], out_hbm.at[pl.ds(0, CHUNK)], ssem[b]).wait()

  return gather_kernel


_gather = _make_gather()


@jax.jit
def kernel(input_ids, tok_emb):
  idx = input_ids.reshape(N_ROWS, CHUNK).astype(jnp.int32)
  out = _gather(idx, tok_emb)
  return out.reshape(input_ids.shape[0], input_ids.shape[1], HIDDEN)
